# Initial kernel scaffold; baseline (speedup 1.0000x reference)
import jax
import jax.numpy as jnp
from jax.experimental import pallas as pl


def kernel(x, edge_index, W_qkv, b_qkv, W_out, b_out):
    raise NotImplementedError("write your pallas kernel here")


# speedup vs baseline: 5.3659x; 5.3659x over previous
import functools

import jax
import jax.numpy as jnp
import numpy as np
from jax import lax
from jax.experimental import pallas as pl
from jax.experimental.pallas import tpu as pltpu
from jax.experimental.pallas import tpu_sc as plsc

N_NODES = 10000
N_EDGES = 320000
D = 128
H = 8
HD = 16
HP = 16

NC = 2
NS = 16
NW = NC * NS
EPW = N_EDGES // NW
EPC = N_EDGES // NC
CH = 80
NCHUNK = EPW // CH
NPC = 5120
NACC = NPC + 8
TRASH = NPC
RPT = NPC // NS
EPT = N_EDGES // NS
NCH2 = EPT // CH

@functools.lru_cache(maxsize=None)
def _sc_mesh():
    return plsc.VectorSubcoreMesh(core_axis_name="c", subcore_axis_name="s")


def _qkv_body(x_ref, w_ref, b_ref, q_ref, k_ref, v_ref):
    x = x_ref[...]
    r = lax.dot_general(x, w_ref[...], (((1,), (1,)), ((), ())),
                        preferred_element_type=jnp.float32)
    r = r + b_ref[...]
    q_ref[...] = r[:, 0:D]
    k_ref[...] = r[:, D:2 * D]
    v_ref[...] = r[:, 2 * D:3 * D]


def _qkv_proj(x, w_qkv, b_qkv):
    blk = 1000
    grid = N_NODES // blk
    return pl.pallas_call(
        _qkv_body,
        grid=(grid,),
        in_specs=[
            pl.BlockSpec((blk, D), lambda i: (i, 0)),
            pl.BlockSpec((3 * D, D), lambda i: (0, 0)),
            pl.BlockSpec((1, 3 * D), lambda i: (0, 0)),
        ],
        out_specs=[pl.BlockSpec((blk, D), lambda i: (i, 0))] * 3,
        out_shape=[jax.ShapeDtypeStruct((N_NODES, D), jnp.float32)] * 3,
    )(x, w_qkv, b_qkv.reshape(1, 3 * D))


def _gather_body(q_hbm, k_hbm, v_hbm, xi_hbm, yi_hbm,
                 qg_hbm, kg_hbm, vg_hbm, xiv, yiv, buf, sem):
    cid = lax.axis_index("c")
    sid = lax.axis_index("s")
    wid = cid * NS + sid

    def body(i, carry):
        base = wid * EPW + i * CH
        pltpu.sync_copy(xi_hbm.at[pl.ds(base, CH)], xiv)
        pltpu.sync_copy(yi_hbm.at[pl.ds(base, CH)], yiv)
        pltpu.async_copy(q_hbm.at[xiv], buf, sem).wait()
        pltpu.sync_copy(buf, qg_hbm.at[pl.ds(base, CH)])
        pltpu.async_copy(k_hbm.at[yiv], buf, sem).wait()
        pltpu.sync_copy(buf, kg_hbm.at[pl.ds(base, CH)])
        pltpu.async_copy(v_hbm.at[yiv], buf, sem).wait()
        pltpu.sync_copy(buf, vg_hbm.at[pl.ds(base, CH)])
        return carry

    lax.fori_loop(0, NCHUNK, body, 0)


@functools.lru_cache(maxsize=None)
def _gather():
    return pl.kernel(
        _gather_body,
        mesh=_sc_mesh(),
        out_type=[jax.ShapeDtypeStruct((N_EDGES, D), jnp.float32)] * 3,
        scratch_types=[
            pltpu.VMEM((CH,), jnp.int32),
            pltpu.VMEM((CH,), jnp.int32),
            pltpu.VMEM((CH, D), jnp.float32),
            pltpu.SemaphoreType.DMA,
        ],
    )


_SCALE = 1.0 / np.sqrt(np.float32(N_NODES))


def _edge_body(qg_ref, kg_ref, vg_ref, uv_ref, ex_ref):
    qg = qg_ref[...]
    kg = kg_ref[...]
    vg = vg_ref[...]
    col = lax.broadcasted_iota(jnp.int32, (D, HP), 1)
    row = lax.broadcasted_iota(jnp.int32, (D, HP), 0)
    m = (col == row // HD).astype(jnp.float32)
    s = lax.dot_general(qg * kg, m, (((1,), (0,)), ((), ())),
                        preferred_element_type=jnp.float32)
    expl = jnp.exp(s * _SCALE)
    w128 = lax.dot_general(expl, m, (((1,), (1,)), ((), ())),
                           preferred_element_type=jnp.float32)
    uv_ref[...] = w128 * vg
    ex_ref[...] = w128


def _edge_math(qg, kg, vg):
    blk = 1000
    grid = N_EDGES // blk
    return pl.pallas_call(
        _edge_body,
        grid=(grid,),
        in_specs=[pl.BlockSpec((blk, D), lambda i: (i, 0))] * 3,
        out_specs=[pl.BlockSpec((blk, D), lambda i: (i, 0)),
                   pl.BlockSpec((blk, D), lambda i: (i, 0))],
        out_shape=[jax.ShapeDtypeStruct((N_EDGES, D), jnp.float32),
                   jax.ShapeDtypeStruct((N_EDGES, D), jnp.float32)],
    )(qg, kg, vg)


ZR = 64


def _scatter_body(uv_hbm, ex_hbm, xi_hbm, zv_hbm, ov_hbm, od_hbm,
                  xiv, xiv2, ubuf, idxb, zv, rv, sem, acc):
    cid = lax.axis_index("c")
    sid = lax.axis_index("s")
    r0 = sid * RPT
    lanes = lax.iota(jnp.int32, 16)
    lo = cid * NPC

    def fill_idx(base):
        for j in range(ZR // 16):
            idxb[pl.ds(j * 16, 16)] = base + j * 16 + lanes

    pltpu.sync_copy(zv_hbm, zv)

    def phase(src_hbm, out_hbm):
        def zbody(i, carry):
            fill_idx(r0 + i * ZR)
            pltpu.sync_copy(zv, acc.at[idxb])
            return carry

        lax.fori_loop(0, RPT // ZR, zbody, 0)
        plsc.subcore_barrier()

        def body(i, carry):
            base = sid * EPT + i * CH
            pltpu.sync_copy(xi_hbm.at[pl.ds(base, CH)], xiv)
            pltpu.sync_copy(src_hbm.at[pl.ds(base, CH)], ubuf)
            for j in range(CH // 16):
                v = xiv[pl.ds(j * 16, 16)]
                rel = v - lo
                ok = (rel >= 0) & (rel < NPC)
                xiv2[pl.ds(j * 16, 16)] = jnp.where(ok, rel, TRASH)
            pltpu.sync_copy(ubuf, acc.at[xiv2], add=True)
            return carry

        lax.fori_loop(0, NCH2, body, 0)
        plsc.subcore_barrier()

        def wbody(i, carry):
            fill_idx(r0 + i * ZR)
            pltpu.async_copy(acc.at[idxb], rv, sem).wait()
            pltpu.sync_copy(rv, out_hbm.at[cid, pl.ds(r0 + i * ZR, ZR)])
            return carry

        lax.fori_loop(0, RPT // ZR, wbody, 0)

    phase(uv_hbm, ov_hbm)
    phase(ex_hbm, od_hbm)


@functools.lru_cache(maxsize=None)
def _scatter():
    return pl.kernel(
        _scatter_body,
        mesh=_sc_mesh(),
        out_type=[jax.ShapeDtypeStruct((NC, NPC, D), jnp.float32),
                  jax.ShapeDtypeStruct((NC, NPC, D), jnp.float32)],
        scratch_types=[
            pltpu.VMEM((CH,), jnp.int32),
            pltpu.VMEM((CH,), jnp.int32),
            pltpu.VMEM((CH, D), jnp.float32),
            pltpu.VMEM((ZR,), jnp.int32),
            pltpu.VMEM((ZR, D), jnp.float32),
            pltpu.VMEM((ZR, D), jnp.float32),
            pltpu.SemaphoreType.DMA,
            pltpu.VMEM_SHARED((NACC, D), jnp.float32),
        ],
    )


def _finish_body(av_ref, ad_ref, w_ref, b_ref, o_ref):
    res = av_ref[...] / (ad_ref[...] + jnp.float32(1e-16))
    out = lax.dot_general(res, w_ref[...], (((1,), (1,)), ((), ())),
                          preferred_element_type=jnp.float32)
    o_ref[...] = out + b_ref[...]


def _finish(av, ad, w_out, b_out):
    blk = 1000
    grid = N_NODES // blk
    return pl.pallas_call(
        _finish_body,
        grid=(grid,),
        in_specs=[
            pl.BlockSpec((blk, D), lambda i: (i, 0)),
            pl.BlockSpec((blk, D), lambda i: (i, 0)),
            pl.BlockSpec((D, D), lambda i: (0, 0)),
            pl.BlockSpec((1, D), lambda i: (0, 0)),
        ],
        out_specs=pl.BlockSpec((blk, D), lambda i: (i, 0)),
        out_shape=jax.ShapeDtypeStruct((N_NODES, D), jnp.float32),
    )(av, ad, w_out, b_out.reshape(1, D))


@jax.jit
def kernel(x, edge_index, W_qkv, b_qkv, W_out, b_out):
    xi = edge_index[0].astype(jnp.int32)
    yi = edge_index[1].astype(jnp.int32)
    q, k, v = _qkv_proj(x, W_qkv, b_qkv)
    qg, kg, vg = _gather()(q, k, v, xi, yi)
    uv, expl = _edge_math(qg, kg, vg)
    zvh = jnp.zeros((ZR, D), jnp.float32)
    ov, od = _scatter()(uv, expl, xi, zvh)
    av = ov.reshape(NC * NPC, D)
    ad = od.reshape(NC * NPC, D)
    return _finish(av, ad, W_out, b_out)


# speedup vs baseline: 7.0921x; 1.3217x over previous
import functools

import jax
import jax.numpy as jnp
import numpy as np
from jax import lax
from jax.experimental import pallas as pl
from jax.experimental.pallas import tpu as pltpu
from jax.experimental.pallas import tpu_sc as plsc

N_NODES = 10000
N_EDGES = 320000
D = 128
H = 8
HD = 16
HP = 16

NC = 2
NS = 16
NW = NC * NS
EPW = N_EDGES // NW
EPC = N_EDGES // NC
CH = 80
NCHUNK = EPW // CH
NPC = 5120
NACC = NPC + 8
TRASH = NPC
RPT = NPC // NS
EPT = N_EDGES // NS
NCH2 = EPT // CH

@functools.lru_cache(maxsize=None)
def _sc_mesh():
    return plsc.VectorSubcoreMesh(core_axis_name="c", subcore_axis_name="s")


def _qkv_body(x_ref, w_ref, b_ref, q_ref, k_ref, v_ref):
    x = x_ref[...]
    r = lax.dot_general(x, w_ref[...], (((1,), (1,)), ((), ())),
                        preferred_element_type=jnp.float32)
    r = r + b_ref[...]
    q_ref[...] = r[:, 0:D]
    k_ref[...] = r[:, D:2 * D]
    v_ref[...] = r[:, 2 * D:3 * D]


def _qkv_proj(x, w_qkv, b_qkv):
    blk = 1000
    grid = N_NODES // blk
    return pl.pallas_call(
        _qkv_body,
        grid=(grid,),
        in_specs=[
            pl.BlockSpec((blk, D), lambda i: (i, 0)),
            pl.BlockSpec((3 * D, D), lambda i: (0, 0)),
            pl.BlockSpec((1, 3 * D), lambda i: (0, 0)),
        ],
        out_specs=[pl.BlockSpec((blk, D), lambda i: (i, 0))] * 3,
        out_shape=[jax.ShapeDtypeStruct((N_NODES, D), jnp.float32)] * 3,
    )(x, w_qkv, b_qkv.reshape(1, 3 * D))


def _gather_body(q_hbm, k_hbm, v_hbm, xi_hbm, yi_hbm,
                 qg_hbm, kg_hbm, vg_hbm, xiv, yiv, qbuf, kbuf, vbuf,
                 semi, semg, semw):
    cid = lax.axis_index("c")
    sid = lax.axis_index("s")
    wid = cid * NS + sid

    def body(i, carry):
        base = wid * EPW + i * CH
        ci = pltpu.async_copy(xi_hbm.at[pl.ds(base, CH)], xiv, semi)
        cy = pltpu.async_copy(yi_hbm.at[pl.ds(base, CH)], yiv, semi)
        ci.wait()
        cy.wait()
        g1 = pltpu.async_copy(q_hbm.at[xiv], qbuf, semg)
        g2 = pltpu.async_copy(k_hbm.at[yiv], kbuf, semg)
        g3 = pltpu.async_copy(v_hbm.at[yiv], vbuf, semg)
        g1.wait()
        w1 = pltpu.async_copy(qbuf, qg_hbm.at[pl.ds(base, CH)], semw)
        g2.wait()
        w2 = pltpu.async_copy(kbuf, kg_hbm.at[pl.ds(base, CH)], semw)
        g3.wait()
        w3 = pltpu.async_copy(vbuf, vg_hbm.at[pl.ds(base, CH)], semw)
        w1.wait()
        w2.wait()
        w3.wait()
        return carry

    lax.fori_loop(0, NCHUNK, body, 0)


@functools.lru_cache(maxsize=None)
def _gather():
    return pl.kernel(
        _gather_body,
        mesh=_sc_mesh(),
        out_type=[jax.ShapeDtypeStruct((N_EDGES, D), jnp.float32)] * 3,
        scratch_types=[
            pltpu.VMEM((CH,), jnp.int32),
            pltpu.VMEM((CH,), jnp.int32),
            pltpu.VMEM((CH, D), jnp.float32),
            pltpu.VMEM((CH, D), jnp.float32),
            pltpu.VMEM((CH, D), jnp.float32),
            pltpu.SemaphoreType.DMA,
            pltpu.SemaphoreType.DMA,
            pltpu.SemaphoreType.DMA,
        ],
    )


_SCALE = 1.0 / np.sqrt(np.float32(N_NODES))


def _edge_body(qg_ref, kg_ref, vg_ref, uv_ref, ex_ref):
    qg = qg_ref[...]
    kg = kg_ref[...]
    vg = vg_ref[...]
    col = lax.broadcasted_iota(jnp.int32, (D, HP), 1)
    row = lax.broadcasted_iota(jnp.int32, (D, HP), 0)
    m = (col == row // HD).astype(jnp.float32)
    s = lax.dot_general(qg * kg, m, (((1,), (0,)), ((), ())),
                        preferred_element_type=jnp.float32)
    expl = jnp.exp(s * _SCALE)
    w128 = lax.dot_general(expl, m, (((1,), (1,)), ((), ())),
                           preferred_element_type=jnp.float32)
    uv_ref[...] = w128 * vg
    ex_ref[...] = w128


def _edge_math(qg, kg, vg):
    blk = 1000
    grid = N_EDGES // blk
    return pl.pallas_call(
        _edge_body,
        grid=(grid,),
        in_specs=[pl.BlockSpec((blk, D), lambda i: (i, 0))] * 3,
        out_specs=[pl.BlockSpec((blk, D), lambda i: (i, 0)),
                   pl.BlockSpec((blk, D), lambda i: (i, 0))],
        out_shape=[jax.ShapeDtypeStruct((N_EDGES, D), jnp.float32),
                   jax.ShapeDtypeStruct((N_EDGES, D), jnp.float32)],
    )(qg, kg, vg)


ZR = 64


def _scatter_body(uv_hbm, ex_hbm, xi_hbm, zv_hbm, ov_hbm, od_hbm,
                  xiv, xiv2, ubuf, idxb, zv, rv, sem, semi, semg, acc):
    cid = lax.axis_index("c")
    sid = lax.axis_index("s")
    r0 = sid * RPT
    lanes = lax.iota(jnp.int32, 16)
    lo = cid * NPC

    def fill_idx(base):
        for j in range(ZR // 16):
            idxb[pl.ds(j * 16, 16)] = base + j * 16 + lanes

    pltpu.sync_copy(zv_hbm, zv)

    def phase(src_hbm, out_hbm):
        def zbody(i, carry):
            fill_idx(r0 + i * ZR)
            pltpu.sync_copy(zv, acc.at[idxb])
            return carry

        lax.fori_loop(0, RPT // ZR, zbody, 0)
        plsc.subcore_barrier()

        def body(i, carry):
            base = sid * EPT + i * CH
            l1 = pltpu.async_copy(xi_hbm.at[pl.ds(base, CH)], xiv, semi)
            l2 = pltpu.async_copy(src_hbm.at[pl.ds(base, CH)], ubuf, semg)
            l1.wait()
            for j in range(CH // 16):
                v = xiv[pl.ds(j * 16, 16)]
                rel = v - lo
                ok = (rel >= 0) & (rel < NPC)
                xiv2[pl.ds(j * 16, 16)] = jnp.where(ok, rel, TRASH)
            l2.wait()
            pltpu.sync_copy(ubuf, acc.at[xiv2], add=True)
            return carry

        lax.fori_loop(0, NCH2, body, 0)
        plsc.subcore_barrier()

        def wbody(i, carry):
            fill_idx(r0 + i * ZR)
            pltpu.async_copy(acc.at[idxb], rv, sem).wait()
            pltpu.sync_copy(rv, out_hbm.at[cid, pl.ds(r0 + i * ZR, ZR)])
            return carry

        lax.fori_loop(0, RPT // ZR, wbody, 0)

    phase(uv_hbm, ov_hbm)
    phase(ex_hbm, od_hbm)


@functools.lru_cache(maxsize=None)
def _scatter():
    return pl.kernel(
        _scatter_body,
        mesh=_sc_mesh(),
        out_type=[jax.ShapeDtypeStruct((NC, NPC, D), jnp.float32),
                  jax.ShapeDtypeStruct((NC, NPC, D), jnp.float32)],
        scratch_types=[
            pltpu.VMEM((CH,), jnp.int32),
            pltpu.VMEM((CH,), jnp.int32),
            pltpu.VMEM((CH, D), jnp.float32),
            pltpu.VMEM((ZR,), jnp.int32),
            pltpu.VMEM((ZR, D), jnp.float32),
            pltpu.VMEM((ZR, D), jnp.float32),
            pltpu.SemaphoreType.DMA,
            pltpu.SemaphoreType.DMA,
            pltpu.SemaphoreType.DMA,
            pltpu.VMEM_SHARED((NACC, D), jnp.float32),
        ],
    )


def _finish_body(av_ref, ad_ref, w_ref, b_ref, o_ref):
    res = av_ref[...] / (ad_ref[...] + jnp.float32(1e-16))
    out = lax.dot_general(res, w_ref[...], (((1,), (1,)), ((), ())),
                          preferred_element_type=jnp.float32)
    o_ref[...] = out + b_ref[...]


def _finish(av, ad, w_out, b_out):
    blk = 1000
    grid = N_NODES // blk
    return pl.pallas_call(
        _finish_body,
        grid=(grid,),
        in_specs=[
            pl.BlockSpec((blk, D), lambda i: (i, 0)),
            pl.BlockSpec((blk, D), lambda i: (i, 0)),
            pl.BlockSpec((D, D), lambda i: (0, 0)),
            pl.BlockSpec((1, D), lambda i: (0, 0)),
        ],
        out_specs=pl.BlockSpec((blk, D), lambda i: (i, 0)),
        out_shape=jax.ShapeDtypeStruct((N_NODES, D), jnp.float32),
    )(av, ad, w_out, b_out.reshape(1, D))


@jax.jit
def kernel(x, edge_index, W_qkv, b_qkv, W_out, b_out):
    xi = edge_index[0].astype(jnp.int32)
    yi = edge_index[1].astype(jnp.int32)
    q, k, v = _qkv_proj(x, W_qkv, b_qkv)
    qg, kg, vg = _gather()(q, k, v, xi, yi)
    uv, expl = _edge_math(qg, kg, vg)
    zvh = jnp.zeros((ZR, D), jnp.float32)
    ov, od = _scatter()(uv, expl, xi, zvh)
    av = ov.reshape(NC * NPC, D)
    ad = od.reshape(NC * NPC, D)
    return _finish(av, ad, W_out, b_out)


# speedup vs baseline: 7.4290x; 1.0475x over previous
import functools

import jax
import jax.numpy as jnp
import numpy as np
from jax import lax
from jax.experimental import pallas as pl
from jax.experimental.pallas import tpu as pltpu
from jax.experimental.pallas import tpu_sc as plsc

N_NODES = 10000
N_EDGES = 320000
D = 128
H = 8
HD = 16
HP = 16

NC = 2
NS = 16
NW = NC * NS
EPW = N_EDGES // NW
EPC = N_EDGES // NC
CH = 80
NCHUNK = EPW // CH
NPC = 5120
NACC = NPC + 8
TRASH = NPC
RPT = NPC // NS
EPT = N_EDGES // NS
NCH2 = EPT // CH

@functools.lru_cache(maxsize=None)
def _sc_mesh():
    return plsc.VectorSubcoreMesh(core_axis_name="c", subcore_axis_name="s")


def _qkv_body(x_ref, w_ref, b_ref, q_ref, k_ref, v_ref):
    x = x_ref[...]
    r = lax.dot_general(x, w_ref[...], (((1,), (1,)), ((), ())),
                        preferred_element_type=jnp.float32)
    r = r + b_ref[...]
    q_ref[...] = r[:, 0:D]
    k_ref[...] = r[:, D:2 * D]
    v_ref[...] = r[:, 2 * D:3 * D]


def _qkv_proj(x, w_qkv, b_qkv):
    blk = 1000
    grid = N_NODES // blk
    return pl.pallas_call(
        _qkv_body,
        grid=(grid,),
        in_specs=[
            pl.BlockSpec((blk, D), lambda i: (i, 0)),
            pl.BlockSpec((3 * D, D), lambda i: (0, 0)),
            pl.BlockSpec((1, 3 * D), lambda i: (0, 0)),
        ],
        out_specs=[pl.BlockSpec((blk, D), lambda i: (i, 0))] * 3,
        out_shape=[jax.ShapeDtypeStruct((N_NODES, D), jnp.float32)] * 3,
    )(x, w_qkv, b_qkv.reshape(1, 3 * D))


def _gather_body(q_hbm, k_hbm, v_hbm, xi_hbm, yi_hbm,
                 qg_hbm, kg_hbm, vg_hbm, xiv, yiv, qbuf, kbuf, vbuf,
                 semi, semg, semw):
    cid = lax.axis_index("c")
    sid = lax.axis_index("s")
    wid = cid * NS + sid
    e0 = wid * EPW

    def fire_idx(c, s):
        base = e0 + c * CH
        pltpu.async_copy(xi_hbm.at[pl.ds(base, CH)], xiv[s], semi[s])
        pltpu.async_copy(yi_hbm.at[pl.ds(base, CH)], yiv[s], semi[s])

    def wait_idx(s):
        pltpu.make_async_copy(xi_hbm.at[pl.ds(0, CH)], xiv[s], semi[s]).wait()
        pltpu.make_async_copy(yi_hbm.at[pl.ds(0, CH)], yiv[s], semi[s]).wait()

    def wait_wb(s):
        pltpu.make_async_copy(qbuf[s], qg_hbm.at[pl.ds(0, CH)], semw[s]).wait()
        pltpu.make_async_copy(kbuf[s], kg_hbm.at[pl.ds(0, CH)], semw[s]).wait()
        pltpu.make_async_copy(vbuf[s], vg_hbm.at[pl.ds(0, CH)], semw[s]).wait()

    def run_chunk(c, s):
        base = e0 + c * CH
        g1 = pltpu.async_copy(q_hbm.at[xiv[s]], qbuf[s], semg[s])
        g2 = pltpu.async_copy(k_hbm.at[yiv[s]], kbuf[s], semg[s])
        g3 = pltpu.async_copy(v_hbm.at[yiv[s]], vbuf[s], semg[s])
        g1.wait()
        pltpu.async_copy(qbuf[s], qg_hbm.at[pl.ds(base, CH)], semw[s])
        g2.wait()
        pltpu.async_copy(kbuf[s], kg_hbm.at[pl.ds(base, CH)], semw[s])
        g3.wait()
        pltpu.async_copy(vbuf[s], vg_hbm.at[pl.ds(base, CH)], semw[s])

    fire_idx(0, 0)
    fire_idx(1, 1)

    def pair(g, carry):
        for b in range(2):
            c = 2 * g + b
            wait_idx(b)

            @pl.when(g >= 1)
            def _drain():
                wait_wb(b)

            run_chunk(c, b)

            @pl.when(c + 2 <= NCHUNK - 1)
            def _prefetch():
                fire_idx(c + 2, b)

        return carry

    lax.fori_loop(0, NCHUNK // 2, pair, 0)

    c = NCHUNK - 1
    wait_idx(0)
    wait_wb(0)
    run_chunk(c, 0)
    wait_wb(0)
    wait_wb(1)


@functools.lru_cache(maxsize=None)
def _gather():
    return pl.kernel(
        _gather_body,
        mesh=_sc_mesh(),
        out_type=[jax.ShapeDtypeStruct((N_EDGES, D), jnp.float32)] * 3,
        scratch_types=[
            [pltpu.VMEM((CH,), jnp.int32)] * 2,
            [pltpu.VMEM((CH,), jnp.int32)] * 2,
            [pltpu.VMEM((CH, D), jnp.float32)] * 2,
            [pltpu.VMEM((CH, D), jnp.float32)] * 2,
            [pltpu.VMEM((CH, D), jnp.float32)] * 2,
            [pltpu.SemaphoreType.DMA] * 2,
            [pltpu.SemaphoreType.DMA] * 2,
            [pltpu.SemaphoreType.DMA] * 2,
        ],
    )


_SCALE = 1.0 / np.sqrt(np.float32(N_NODES))


def _edge_body(qg_ref, kg_ref, vg_ref, uv_ref, ex_ref):
    qg = qg_ref[...]
    kg = kg_ref[...]
    vg = vg_ref[...]
    col = lax.broadcasted_iota(jnp.int32, (D, HP), 1)
    row = lax.broadcasted_iota(jnp.int32, (D, HP), 0)
    m = (col == row // HD).astype(jnp.float32)
    s = lax.dot_general(qg * kg, m, (((1,), (0,)), ((), ())),
                        preferred_element_type=jnp.float32)
    expl = jnp.exp(s * _SCALE)
    w128 = lax.dot_general(expl, m, (((1,), (1,)), ((), ())),
                           preferred_element_type=jnp.float32)
    uv_ref[...] = w128 * vg
    ex_ref[...] = w128


def _edge_math(qg, kg, vg):
    blk = 1000
    grid = N_EDGES // blk
    return pl.pallas_call(
        _edge_body,
        grid=(grid,),
        in_specs=[pl.BlockSpec((blk, D), lambda i: (i, 0))] * 3,
        out_specs=[pl.BlockSpec((blk, D), lambda i: (i, 0)),
                   pl.BlockSpec((blk, D), lambda i: (i, 0))],
        out_shape=[jax.ShapeDtypeStruct((N_EDGES, D), jnp.float32),
                   jax.ShapeDtypeStruct((N_EDGES, D), jnp.float32)],
    )(qg, kg, vg)


ZR = 64


def _scatter_body(uv_hbm, ex_hbm, xi_hbm, zv_hbm, ov_hbm, od_hbm,
                  xiv, xiv2, ubuf, idxb, zv, rv, sem, semi, semg, acc):
    cid = lax.axis_index("c")
    sid = lax.axis_index("s")
    r0 = sid * RPT
    lanes = lax.iota(jnp.int32, 16)
    lo = cid * NPC

    def fill_idx(base):
        for j in range(ZR // 16):
            idxb[pl.ds(j * 16, 16)] = base + j * 16 + lanes

    pltpu.sync_copy(zv_hbm, zv)

    def phase(src_hbm, out_hbm):
        def zbody(i, carry):
            fill_idx(r0 + i * ZR)
            pltpu.sync_copy(zv, acc.at[idxb])
            return carry

        lax.fori_loop(0, RPT // ZR, zbody, 0)
        plsc.subcore_barrier()

        def body(i, carry):
            base = sid * EPT + i * CH
            l1 = pltpu.async_copy(xi_hbm.at[pl.ds(base, CH)], xiv, semi)
            l2 = pltpu.async_copy(src_hbm.at[pl.ds(base, CH)], ubuf, semg)
            l1.wait()
            for j in range(CH // 16):
                v = xiv[pl.ds(j * 16, 16)]
                rel = v - lo
                ok = (rel >= 0) & (rel < NPC)
                xiv2[pl.ds(j * 16, 16)] = jnp.where(ok, rel, TRASH)
            l2.wait()
            pltpu.sync_copy(ubuf, acc.at[xiv2], add=True)
            return carry

        lax.fori_loop(0, NCH2, body, 0)
        plsc.subcore_barrier()

        def wbody(i, carry):
            fill_idx(r0 + i * ZR)
            pltpu.async_copy(acc.at[idxb], rv, sem).wait()
            pltpu.sync_copy(rv, out_hbm.at[cid, pl.ds(r0 + i * ZR, ZR)])
            return carry

        lax.fori_loop(0, RPT // ZR, wbody, 0)

    phase(uv_hbm, ov_hbm)
    phase(ex_hbm, od_hbm)


@functools.lru_cache(maxsize=None)
def _scatter():
    return pl.kernel(
        _scatter_body,
        mesh=_sc_mesh(),
        out_type=[jax.ShapeDtypeStruct((NC, NPC, D), jnp.float32),
                  jax.ShapeDtypeStruct((NC, NPC, D), jnp.float32)],
        scratch_types=[
            pltpu.VMEM((CH,), jnp.int32),
            pltpu.VMEM((CH,), jnp.int32),
            pltpu.VMEM((CH, D), jnp.float32),
            pltpu.VMEM((ZR,), jnp.int32),
            pltpu.VMEM((ZR, D), jnp.float32),
            pltpu.VMEM((ZR, D), jnp.float32),
            pltpu.SemaphoreType.DMA,
            pltpu.SemaphoreType.DMA,
            pltpu.SemaphoreType.DMA,
            pltpu.VMEM_SHARED((NACC, D), jnp.float32),
        ],
    )


def _finish_body(av_ref, ad_ref, w_ref, b_ref, o_ref):
    res = av_ref[...] / (ad_ref[...] + jnp.float32(1e-16))
    out = lax.dot_general(res, w_ref[...], (((1,), (1,)), ((), ())),
                          preferred_element_type=jnp.float32)
    o_ref[...] = out + b_ref[...]


def _finish(av, ad, w_out, b_out):
    blk = 1000
    grid = N_NODES // blk
    return pl.pallas_call(
        _finish_body,
        grid=(grid,),
        in_specs=[
            pl.BlockSpec((blk, D), lambda i: (i, 0)),
            pl.BlockSpec((blk, D), lambda i: (i, 0)),
            pl.BlockSpec((D, D), lambda i: (0, 0)),
            pl.BlockSpec((1, D), lambda i: (0, 0)),
        ],
        out_specs=pl.BlockSpec((blk, D), lambda i: (i, 0)),
        out_shape=jax.ShapeDtypeStruct((N_NODES, D), jnp.float32),
    )(av, ad, w_out, b_out.reshape(1, D))


@jax.jit
def kernel(x, edge_index, W_qkv, b_qkv, W_out, b_out):
    xi = edge_index[0].astype(jnp.int32)
    yi = edge_index[1].astype(jnp.int32)
    q, k, v = _qkv_proj(x, W_qkv, b_qkv)
    qg, kg, vg = _gather()(q, k, v, xi, yi)
    uv, expl = _edge_math(qg, kg, vg)
    zvh = jnp.zeros((ZR, D), jnp.float32)
    ov, od = _scatter()(uv, expl, xi, zvh)
    av = ov.reshape(NC * NPC, D)
    ad = od.reshape(NC * NPC, D)
    return _finish(av, ad, W_out, b_out)
